# Initial kernel scaffold; baseline (speedup 1.0000x reference)
import jax
import jax.numpy as jnp
from jax.experimental import pallas as pl


def kernel(input, codes, codebook):
    raise NotImplementedError("write your pallas kernel here")


# speedup vs baseline: 27.4758x; 27.4758x over previous
import functools

import jax
import jax.numpy as jnp
from jax import lax
from jax.experimental import pallas as pl
from jax.experimental.pallas import tpu as pltpu, tpu_sc as plsc

B, L = 4096, 50
DIM = 128
BLOCK_LEN = 4
N_BLOCKS = DIM // BLOCK_LEN
N_CODES = 16

N = B * L
NC, NS = 2, 16
NW = NC * NS
ROWS_PER_W = N // NW
CHUNK = 128
NCHUNK = ROWS_PER_W // CHUNK
PACK = 4


def _decode_body(idx_hbm, codes_hbm, cb_hbm, out_hbm,
                 idx_v, q_v, codes_v, out_v, cb_v, rsplat_v, sem):
    wid = lax.axis_index("s") * NC + lax.axis_index("c")
    base = wid * ROWS_PER_W

    pltpu.sync_copy(cb_hbm, cb_v)

    @pl.loop(0, NCHUNK)
    def chunk_body(g):
        cbase = base + g * CHUNK
        pltpu.sync_copy(idx_hbm.at[pl.ds(cbase, CHUNK)], idx_v)

        @pl.loop(0, CHUNK // 16)
        def qbody(j):
            q_v[pl.ds(j * 16, 16)] = idx_v[pl.ds(j * 16, 16)] >> 2

        pltpu.async_copy(codes_hbm.at[q_v], codes_v, sem).wait()

        rsplat_v[...] = jnp.zeros((16,), jnp.int32)

        @pl.loop(0, CHUNK)
        def row_body(r):
            lane = lax.iota(jnp.int32, 16)
            rep = lane >> 2
            pos = lane & 3
            rvec = rsplat_v[...]
            mvec = plsc.load_gather(idx_v, [rvec]) & 3
            col0 = mvec << 5
            for t in range(DIM // 16):
                code = plsc.load_gather(codes_v, [rvec, col0 + (t * 4 + rep)])
                vals = plsc.load_gather(cb_v, [(code << 2) + pos])
                out_v[r, pl.ds(t * 16, 16)] = vals
            rsplat_v[...] = rvec + 1

        pltpu.sync_copy(out_v, out_hbm.at[pl.ds(cbase, CHUNK)])


@jax.jit
def _decode(idx, codes2, cb_flat):
    mesh = plsc.VectorSubcoreMesh(core_axis_name="c", subcore_axis_name="s")
    return pl.kernel(
        _decode_body,
        out_type=jax.ShapeDtypeStruct((N, DIM), jnp.float32),
        mesh=mesh,
        compiler_params=pltpu.CompilerParams(needs_layout_passes=False),
        scratch_types=[
            pltpu.VMEM((CHUNK,), jnp.int32),
            pltpu.VMEM((CHUNK,), jnp.int32),
            pltpu.VMEM((CHUNK, DIM), jnp.int32),
            pltpu.VMEM((CHUNK, DIM), jnp.float32),
            pltpu.VMEM((N_CODES * BLOCK_LEN,), jnp.float32),
            pltpu.VMEM((16,), jnp.int32),
            pltpu.SemaphoreType.DMA,
        ],
    )(idx, codes2, cb_flat)


def kernel(input, codes, codebook):
    idx = input.reshape(-1).astype(jnp.int32)
    codes2 = codes.reshape(codes.shape[0] // PACK, N_BLOCKS * PACK)
    cb_flat = codebook.reshape(-1)
    out = _decode(idx, codes2, cb_flat)
    return out.reshape(input.shape + (DIM,))


# speedup vs baseline: 30.5216x; 1.1109x over previous
import functools

import jax
import jax.numpy as jnp
from jax import lax
from jax.experimental import pallas as pl
from jax.experimental.pallas import tpu as pltpu, tpu_sc as plsc

B, L = 4096, 50
DIM = 128
BLOCK_LEN = 4
N_BLOCKS = DIM // BLOCK_LEN
N_CODES = 16

N = B * L
NC, NS = 2, 16
NW = NC * NS
ROWS_PER_W = N // NW
CHUNK = 128
NCHUNK = ROWS_PER_W // CHUNK
PACK = 4
NBUF = 2


def _decode_body(idx_hbm, codes_hbm, cb_hbm, out_hbm,
                 idx_v, q_v, codes_v, out_v, cb_v, rsplat_v, gsem, osem):
    wid = lax.axis_index("s") * NC + lax.axis_index("c")
    base = wid * ROWS_PER_W

    pltpu.sync_copy(cb_hbm, cb_v)

    def start_gather(b, cbase):
        pltpu.sync_copy(idx_hbm.at[pl.ds(cbase, CHUNK)], idx_v[b])

        @pl.loop(0, CHUNK // 16)
        def qbody(j):
            q_v[b][pl.ds(j * 16, 16)] = idx_v[b][pl.ds(j * 16, 16)] >> 2

        pltpu.async_copy(codes_hbm.at[q_v[b]], codes_v[b], gsem[b])

    for b in range(NBUF):
        start_gather(b, base + b * CHUNK)

    @pl.loop(0, NCHUNK, step=NBUF)
    def chunk_body(g):
        for b in range(NBUF):
            cbase = base + (g + b) * CHUNK
            pltpu.make_async_copy(codes_hbm.at[q_v[b]], codes_v[b],
                                  gsem[b]).wait()
            @pl.when(g > 0)
            def _wait_out():
                pltpu.make_async_copy(
                    out_v[b], out_hbm.at[pl.ds(cbase - NBUF * CHUNK, CHUNK)],
                    osem[b]).wait()

            rsplat_v[...] = jnp.zeros((16,), jnp.int32)

            @pl.loop(0, CHUNK)
            def row_body(r):
                lane = lax.iota(jnp.int32, 16)
                rep = lane >> 2
                pos = lane & 3
                rvec = rsplat_v[...]
                mvec = plsc.load_gather(idx_v[b], [rvec]) & 3
                col0 = mvec << 5
                for t in range(DIM // 16):
                    code = plsc.load_gather(codes_v[b],
                                            [rvec, col0 + (t * 4 + rep)])
                    vals = plsc.load_gather(cb_v, [(code << 2) + pos])
                    out_v[b][r, pl.ds(t * 16, 16)] = vals
                rsplat_v[...] = rvec + 1

            @pl.when(g + b + NBUF < NCHUNK)
            def _next_gather():
                start_gather(b, cbase + NBUF * CHUNK)

            pltpu.async_copy(out_v[b], out_hbm.at[pl.ds(cbase, CHUNK)],
                             osem[b])

    for b in range(NBUF):
        last = base + (NCHUNK - NBUF + b) * CHUNK
        pltpu.make_async_copy(out_v[b], out_hbm.at[pl.ds(last, CHUNK)],
                              osem[b]).wait()


@jax.jit
def _decode(idx, codes2, cb_flat):
    mesh = plsc.VectorSubcoreMesh(core_axis_name="c", subcore_axis_name="s")
    return pl.kernel(
        _decode_body,
        out_type=jax.ShapeDtypeStruct((N, DIM), jnp.float32),
        mesh=mesh,
        compiler_params=pltpu.CompilerParams(needs_layout_passes=False),
        scratch_types=[
            [pltpu.VMEM((CHUNK,), jnp.int32) for _ in range(NBUF)],
            [pltpu.VMEM((CHUNK,), jnp.int32) for _ in range(NBUF)],
            [pltpu.VMEM((CHUNK, DIM), jnp.int32) for _ in range(NBUF)],
            [pltpu.VMEM((CHUNK, DIM), jnp.float32) for _ in range(NBUF)],
            pltpu.VMEM((N_CODES * BLOCK_LEN,), jnp.float32),
            pltpu.VMEM((16,), jnp.int32),
            [pltpu.SemaphoreType.DMA for _ in range(NBUF)],
            [pltpu.SemaphoreType.DMA for _ in range(NBUF)],
        ],
    )(idx, codes2, cb_flat)


def kernel(input, codes, codebook):
    idx = input.reshape(-1).astype(jnp.int32)
    codes2 = codes.reshape(codes.shape[0] // PACK, N_BLOCKS * PACK)
    cb_flat = codebook.reshape(-1)
    out = _decode(idx, codes2, cb_flat)
    return out.reshape(input.shape + (DIM,))


# speedup vs baseline: 58.7802x; 1.9259x over previous
import functools

import jax
import jax.numpy as jnp
from jax import lax
from jax.experimental import pallas as pl
from jax.experimental.pallas import tpu as pltpu, tpu_sc as plsc

B, L = 4096, 50
DIM = 128
BLOCK_LEN = 4
N_BLOCKS = DIM // BLOCK_LEN
N_CODES = 16

N = B * L
NC, NS = 2, 16
NW = NC * NS
ROWS_PER_W = N // NW
CHUNK = 128
NCHUNK = ROWS_PER_W // CHUNK
PACK = 4
NBUF = 2


def _decode_body(idx_hbm, codes_hbm, cb_hbm, out_hbm,
                 idx_v, q_v, codes_v, out_v, cb_v, rsplat_v, gsem, osem):
    wid = lax.axis_index("s") * NC + lax.axis_index("c")
    base = wid * ROWS_PER_W

    pltpu.sync_copy(cb_hbm, cb_v)

    def start_gather(b, cbase):
        pltpu.sync_copy(idx_hbm.at[pl.ds(cbase, CHUNK)], idx_v[b])

        @pl.loop(0, CHUNK // 16)
        def qbody(j):
            q_v[b][pl.ds(j * 16, 16)] = idx_v[b][pl.ds(j * 16, 16)] >> 2

        pltpu.async_copy(codes_hbm.at[q_v[b]], codes_v[b], gsem[b])

    for b in range(NBUF):
        start_gather(b, base + b * CHUNK)

    @pl.loop(0, NCHUNK, step=NBUF)
    def chunk_body(g):
        for b in range(NBUF):
            cbase = base + (g + b) * CHUNK
            pltpu.make_async_copy(codes_hbm.at[q_v[b]], codes_v[b],
                                  gsem[b]).wait()
            @pl.when(g > 0)
            def _wait_out():
                pltpu.make_async_copy(
                    out_v[b], out_hbm.at[pl.ds(cbase - NBUF * CHUNK, CHUNK)],
                    osem[b]).wait()

            rsplat_v[...] = jnp.zeros((16,), jnp.int32)

            @pl.loop(0, CHUNK)
            def row_body(r):
                lane = lax.iota(jnp.int32, 16)
                rep = lane >> 2
                pos = lane & 3
                rvec = rsplat_v[...]
                mvec = plsc.load_gather(idx_v[b], [rvec]) & 3
                col0 = mvec << 5
                codes_t = [plsc.load_gather(codes_v[b],
                                            [rvec, col0 + (t * 4 + rep)])
                           for t in range(DIM // 16)]
                vals_t = [plsc.load_gather(cb_v, [(c << 2) + pos])
                          for c in codes_t]
                for t in range(DIM // 16):
                    out_v[b][r, pl.ds(t * 16, 16)] = vals_t[t]
                rsplat_v[...] = rvec + 1

            @pl.when(g + b + NBUF < NCHUNK)
            def _next_gather():
                start_gather(b, cbase + NBUF * CHUNK)

            pltpu.async_copy(out_v[b], out_hbm.at[pl.ds(cbase, CHUNK)],
                             osem[b])

    for b in range(NBUF):
        last = base + (NCHUNK - NBUF + b) * CHUNK
        pltpu.make_async_copy(out_v[b], out_hbm.at[pl.ds(last, CHUNK)],
                              osem[b]).wait()


@jax.jit
def _decode(idx, codes2, cb_flat):
    mesh = plsc.VectorSubcoreMesh(core_axis_name="c", subcore_axis_name="s")
    return pl.kernel(
        _decode_body,
        out_type=jax.ShapeDtypeStruct((N, DIM), jnp.float32),
        mesh=mesh,
        compiler_params=pltpu.CompilerParams(needs_layout_passes=False),
        scratch_types=[
            [pltpu.VMEM((CHUNK,), jnp.int32) for _ in range(NBUF)],
            [pltpu.VMEM((CHUNK,), jnp.int32) for _ in range(NBUF)],
            [pltpu.VMEM((CHUNK, DIM), jnp.int32) for _ in range(NBUF)],
            [pltpu.VMEM((CHUNK, DIM), jnp.float32) for _ in range(NBUF)],
            pltpu.VMEM((N_CODES * BLOCK_LEN,), jnp.float32),
            pltpu.VMEM((16,), jnp.int32),
            [pltpu.SemaphoreType.DMA for _ in range(NBUF)],
            [pltpu.SemaphoreType.DMA for _ in range(NBUF)],
        ],
    )(idx, codes2, cb_flat)


def kernel(input, codes, codebook):
    idx = input.reshape(-1).astype(jnp.int32)
    codes2 = codes.reshape(codes.shape[0] // PACK, N_BLOCKS * PACK)
    cb_flat = codebook.reshape(-1)
    out = _decode(idx, codes2, cb_flat)
    return out.reshape(input.shape + (DIM,))
